# Initial kernel scaffold; baseline (speedup 1.0000x reference)
import jax
import jax.numpy as jnp
from jax.experimental import pallas as pl


def kernel(inputs, embedding_table):
    raise NotImplementedError("write your pallas kernel here")


# speedup vs baseline: 2.4404x; 2.4404x over previous
import functools

import jax
import jax.numpy as jnp
from jax import lax
from jax.experimental import pallas as pl
from jax.experimental.pallas import tpu as pltpu
from jax.experimental.pallas import tpu_sc as plsc


def _gather_body(idx_hbm, table_hbm, out_hbm, idx_v, rows_v, sem,
                 *, per_w, chunk, n_chunks, nc):
    wid = lax.axis_index("s") * nc + lax.axis_index("c")
    w_base = wid * per_w

    def one_chunk(i, carry):
        base = w_base + i * chunk
        pltpu.sync_copy(idx_hbm.at[pl.ds(base, chunk)], idx_v)
        pltpu.async_copy(table_hbm.at[idx_v], rows_v, sem).wait()
        pltpu.sync_copy(rows_v, out_hbm.at[pl.ds(base, chunk)])
        return carry

    lax.fori_loop(0, n_chunks, one_chunk, 0)


def _make_sc_gather(n_rows, d):
    info = plsc.get_sparse_core_info()
    nc, ns = info.num_cores, info.num_subcores
    nw = nc * ns
    assert n_rows % (8 * nw) == 0
    per_w = n_rows // nw
    chunk = 1024
    while per_w % chunk != 0:
        chunk //= 2
    n_chunks = per_w // chunk

    mesh = plsc.VectorSubcoreMesh(core_axis_name="c", subcore_axis_name="s")
    return pl.kernel(
        functools.partial(_gather_body, per_w=per_w, chunk=chunk,
                          n_chunks=n_chunks, nc=nc),
        mesh=mesh,
        out_type=jax.ShapeDtypeStruct((n_rows, d), jnp.float32),
        scratch_types=[
            pltpu.VMEM((chunk,), jnp.int32),
            pltpu.VMEM((chunk, d), jnp.float32),
            pltpu.SemaphoreType.DMA,
        ],
        compiler_params=pltpu.CompilerParams(use_tc_tiling_on_sc=False),
    )


def kernel(inputs, embedding_table):
    b, t = inputs.shape
    v, d = embedding_table.shape
    idx = inputs.reshape(-1).astype(jnp.int32)
    out = _make_sc_gather(b * t, d)(idx, embedding_table)
    return out.reshape(b, t, d)


# speedup vs baseline: 12.1089x; 4.9618x over previous
import functools

import jax
import jax.numpy as jnp
from jax import lax
from jax.experimental import pallas as pl
from jax.experimental.pallas import tpu as pltpu
from jax.experimental.pallas import tpu_sc as plsc


def _gather_body(idx_hbm, table_hbm, out_hbm, table_v, idx_v, rows_v, sem,
                 *, per_w, chunk, n_chunks, nc):
    sid = lax.axis_index("s")
    wid = sid * nc + lax.axis_index("c")
    w_base = wid * per_w

    @pl.when(sid == 0)
    def _():
        pltpu.sync_copy(table_hbm, table_v)

    plsc.subcore_barrier()

    def one_chunk(i, carry):
        base = w_base + i * chunk
        pltpu.sync_copy(idx_hbm.at[pl.ds(base, chunk)], idx_v)
        pltpu.async_copy(table_v.at[idx_v], rows_v, sem).wait()
        pltpu.sync_copy(rows_v, out_hbm.at[pl.ds(base, chunk)])
        return carry

    lax.fori_loop(0, n_chunks, one_chunk, 0)


def _make_sc_gather(n_rows, d):
    info = plsc.get_sparse_core_info()
    nc, ns = info.num_cores, info.num_subcores
    nw = nc * ns
    assert n_rows % (8 * nw) == 0
    per_w = n_rows // nw
    chunk = 1024
    while per_w % chunk != 0:
        chunk //= 2
    n_chunks = per_w // chunk

    mesh = plsc.VectorSubcoreMesh(core_axis_name="c", subcore_axis_name="s")
    return pl.kernel(
        functools.partial(_gather_body, per_w=per_w, chunk=chunk,
                          n_chunks=n_chunks, nc=nc),
        mesh=mesh,
        out_type=jax.ShapeDtypeStruct((n_rows, d), jnp.float32),
        scratch_types=[
            pltpu.VMEM_SHARED((13, d), jnp.float32),
            pltpu.VMEM((chunk,), jnp.int32),
            pltpu.VMEM((chunk, d), jnp.float32),
            pltpu.SemaphoreType.DMA,
        ],
        compiler_params=pltpu.CompilerParams(use_tc_tiling_on_sc=False),
    )


def kernel(inputs, embedding_table):
    b, t = inputs.shape
    v, d = embedding_table.shape
    idx = inputs.reshape(-1).astype(jnp.int32)
    out = _make_sc_gather(b * t, d)(idx, embedding_table)
    return out.reshape(b, t, d)


# speedup vs baseline: 13.4072x; 1.1072x over previous
import functools

import jax
import jax.numpy as jnp
from jax import lax
from jax.experimental import pallas as pl
from jax.experimental.pallas import tpu as pltpu
from jax.experimental.pallas import tpu_sc as plsc


def _gather_body(idx_hbm, table_hbm, out_hbm, table_sh,
                 idx_v0, idx_v1, rows_v0, rows_v1,
                 s_i0, s_i1, s_o0, s_o1, s_g,
                 *, per_w, chunk, n_chunks, nc):
    idx_v = (idx_v0, idx_v1)
    rows_v = (rows_v0, rows_v1)
    s_i = (s_i0, s_i1)
    s_o = (s_o0, s_o1)

    sid = lax.axis_index("s")
    wid = sid * nc + lax.axis_index("c")
    base0 = wid * per_w

    pltpu.sync_copy(table_hbm, table_sh.at[sid])
    plsc.subcore_barrier()
    table_my = table_sh.at[sid]

    for b in (0, 1):
        pltpu.async_copy(idx_hbm.at[pl.ds(base0 + b * chunk, chunk)],
                         idx_v[b], s_i[b])

    def one_pair(io, carry):
        for b in (0, 1):
            i = io * 2 + b
            base = base0 + i * chunk
            pltpu.make_async_copy(
                idx_hbm.at[pl.ds(base, chunk)], idx_v[b], s_i[b]).wait()

            @pl.when(io >= 1)
            def _():
                pltpu.make_async_copy(
                    rows_v[b], out_hbm.at[pl.ds(base, chunk)], s_o[b]).wait()

            handles = [
                pltpu.async_copy(table_my.at[idx_v[b].at[r]],
                                 rows_v[b].at[r], s_g)
                for r in range(chunk)
            ]
            for h in handles:
                h.wait()

            pltpu.async_copy(rows_v[b], out_hbm.at[pl.ds(base, chunk)],
                             s_o[b])

            @pl.when(i + 2 < n_chunks)
            def _():
                pltpu.async_copy(
                    idx_hbm.at[pl.ds(base + 2 * chunk, chunk)],
                    idx_v[b], s_i[b])
        return carry

    lax.fori_loop(0, n_chunks // 2, one_pair, 0)

    for b in (0, 1):
        pltpu.make_async_copy(
            rows_v[b], out_hbm.at[pl.ds(base0, chunk)], s_o[b]).wait()


def _make_sc_gather(n_b, n_t, v, d):
    info = plsc.get_sparse_core_info()
    nc, ns = info.num_cores, info.num_subcores
    nw = nc * ns
    assert n_b % nw == 0
    per_w = n_b // nw
    chunk = 8
    while per_w % (2 * chunk) != 0:
        chunk //= 2
    n_chunks = per_w // chunk

    mesh = plsc.VectorSubcoreMesh(core_axis_name="c", subcore_axis_name="s")
    return pl.kernel(
        functools.partial(_gather_body, per_w=per_w, chunk=chunk,
                          n_chunks=n_chunks, nc=nc),
        mesh=mesh,
        out_type=jax.ShapeDtypeStruct((n_b, n_t, d), jnp.float32),
        scratch_types=[
            pltpu.VMEM_SHARED((ns, v, d), jnp.float32),
            pltpu.VMEM((chunk, n_t), jnp.int32),
            pltpu.VMEM((chunk, n_t), jnp.int32),
            pltpu.VMEM((chunk, n_t, d), jnp.float32),
            pltpu.VMEM((chunk, n_t, d), jnp.float32),
            pltpu.SemaphoreType.DMA,
            pltpu.SemaphoreType.DMA,
            pltpu.SemaphoreType.DMA,
            pltpu.SemaphoreType.DMA,
            pltpu.SemaphoreType.DMA,
        ],
        compiler_params=pltpu.CompilerParams(use_tc_tiling_on_sc=False),
    )


def kernel(inputs, embedding_table):
    n_b, n_t = inputs.shape
    v, d = embedding_table.shape
    idx = inputs.astype(jnp.int32)
    return _make_sc_gather(n_b, n_t, v, d)(idx, embedding_table)


# speedup vs baseline: 15.6007x; 1.1636x over previous
import functools

import jax
import jax.numpy as jnp
from jax import lax
from jax.experimental import pallas as pl
from jax.experimental.pallas import tpu as pltpu
from jax.experimental.pallas import tpu_sc as plsc

_LANES = 16


def _seg_body(idx_hbm, tab_hbm, out_hbm, idx_v, tab_v, st0, st1, sem0, sem1,
              *, n_t, n_b, segs_per_w, nc):
    n_bt = n_b // 128
    chunk_bt = 16
    chunk_w = chunk_bt * 1024
    n_chunks = n_bt // chunk_bt
    groups = chunk_w // 128

    wid = lax.axis_index("s") * nc + lax.axis_index("c")
    pltpu.sync_copy(tab_hbm, tab_v)
    s0 = wid * segs_per_w

    def seg_loop(sl, carry):
        s = s0 + sl
        t = s // 8
        dt = s % 8
        pltpu.sync_copy(idx_hbm.at[t], idx_v)
        seg_base = t * (64 * n_b) + dt * (8 * n_b)
        d_base = dt * 128

        def pair_loop(p, carry2):
            for b, st, sem in ((0, st0, sem0), (1, st1, sem1)):
                c = p * 2 + b

                @pl.when((sl > 0) | (p > 0))
                def _():
                    pltpu.make_async_copy(
                        st, out_hbm.at[pl.ds(0, chunk_w)], sem).wait()

                def grp_loop(g, carry3):
                    gg = c * groups + g
                    idxv = idx_v[pl.ds(gg * _LANES, _LANES)]
                    off0 = (g // 8) * 1024 + (g % 8) * _LANES
                    for di in range(8):
                        addr = idxv + (d_base + di * _LANES)
                        val = plsc.load_gather(tab_v, [addr])
                        st[pl.ds(off0 + di * 128, _LANES)] = val
                    return carry3

                lax.fori_loop(0, groups, grp_loop, 0)
                pltpu.async_copy(
                    st, out_hbm.at[pl.ds(seg_base + c * chunk_w, chunk_w)],
                    sem)
            return carry2

        lax.fori_loop(0, n_chunks // 2, pair_loop, 0)
        return carry

    lax.fori_loop(0, segs_per_w, seg_loop, 0)
    for st, sem in ((st0, sem0), (st1, sem1)):
        pltpu.make_async_copy(st, out_hbm.at[pl.ds(0, chunk_w)], sem).wait()


def _make_sc_kernel(n_b, n_t, d):
    info = plsc.get_sparse_core_info()
    nc, ns = info.num_cores, info.num_subcores
    nw = nc * ns
    n_seg = n_t * (d // 8)
    assert n_seg % nw == 0 and n_b % 2048 == 0
    segs_per_w = n_seg // nw
    chunk_w = 16 * 1024

    mesh = plsc.VectorSubcoreMesh(core_axis_name="c", subcore_axis_name="s")
    return pl.kernel(
        functools.partial(_seg_body, n_t=n_t, n_b=n_b,
                          segs_per_w=segs_per_w, nc=nc),
        mesh=mesh,
        out_type=jax.ShapeDtypeStruct((n_b * n_t * d,), jnp.float32),
        scratch_types=[
            pltpu.VMEM((n_b,), jnp.int32),
            pltpu.VMEM((d * _LANES,), jnp.float32),
            pltpu.VMEM((chunk_w,), jnp.float32),
            pltpu.VMEM((chunk_w,), jnp.float32),
            pltpu.SemaphoreType.DMA,
            pltpu.SemaphoreType.DMA,
        ],
        compiler_params=pltpu.CompilerParams(use_tc_tiling_on_sc=False,
                                             needs_layout_passes=False),
    )


def kernel(inputs, embedding_table):
    n_b, n_t = inputs.shape
    v, d = embedding_table.shape
    idx_t = inputs.T.astype(jnp.int32)
    tab_t = jnp.pad(embedding_table.T.astype(jnp.float32),
                    ((0, 0), (0, _LANES - v))).reshape(-1)
    flat = _make_sc_kernel(n_b, n_t, d)(idx_t, tab_t)
    out5 = flat.reshape(n_t, d // 8, n_b // 128, 8, 128)
    return out5.transpose(2, 4, 0, 1, 3).reshape(n_b, n_t, d)


# speedup vs baseline: 76.3799x; 4.8959x over previous
import functools

import jax
import jax.numpy as jnp
from jax import lax
from jax.experimental import pallas as pl
from jax.experimental.pallas import tpu as pltpu
from jax.experimental.pallas import tpu_sc as plsc

_LANES = 16


def _seg_body(idx_hbm, tab_hbm, out_hbm, idx_v, tab_v, st0, st1, sem0, sem1,
              *, n_t, n_b, segs_per_w, nc):
    n_bt = n_b // 128
    chunk_bt = 16
    chunk_w = chunk_bt * 1024
    n_chunks = n_bt // chunk_bt
    groups = chunk_w // 128

    wid = lax.axis_index("s") * nc + lax.axis_index("c")
    pltpu.sync_copy(tab_hbm, tab_v)
    s0 = wid * segs_per_w

    def seg_loop(sl, carry):
        s = s0 + sl
        t = s // 8
        dt = s % 8
        pltpu.sync_copy(idx_hbm.at[t], idx_v)
        seg_base = t * (64 * n_b) + dt * (8 * n_b)
        d_base = dt * 128
        zero16 = jnp.zeros((_LANES,), jnp.int32)
        bvecs = [zero16 + (d_base + di * _LANES) for di in range(8)]

        def pair_loop(p, carry2):
            for b, st, sem in ((0, st0, sem0), (1, st1, sem1)):
                c = p * 2 + b

                @pl.when((sl > 0) | (p > 0))
                def _():
                    pltpu.make_async_copy(
                        st, out_hbm.at[pl.ds(0, chunk_w)], sem).wait()

                @plsc.parallel_loop(0, groups, unroll=4)
                def _(g):
                    gg = c * groups + g
                    idxv = idx_v[pl.ds(gg * _LANES, _LANES)]
                    off0 = (g // 8) * 1024 + (g % 8) * _LANES
                    for di in range(8):
                        val = plsc.load_gather(tab_v, [idxv + bvecs[di]])
                        st[pl.ds(off0 + di * 128, _LANES)] = val
                pltpu.async_copy(
                    st, out_hbm.at[pl.ds(seg_base + c * chunk_w, chunk_w)],
                    sem)
            return carry2

        lax.fori_loop(0, n_chunks // 2, pair_loop, 0)
        return carry

    lax.fori_loop(0, segs_per_w, seg_loop, 0)
    for st, sem in ((st0, sem0), (st1, sem1)):
        pltpu.make_async_copy(st, out_hbm.at[pl.ds(0, chunk_w)], sem).wait()


def _make_sc_kernel(n_b, n_t, d):
    info = plsc.get_sparse_core_info()
    nc, ns = info.num_cores, info.num_subcores
    nw = nc * ns
    n_seg = n_t * (d // 8)
    assert n_seg % nw == 0 and n_b % 2048 == 0
    segs_per_w = n_seg // nw
    chunk_w = 16 * 1024

    mesh = plsc.VectorSubcoreMesh(core_axis_name="c", subcore_axis_name="s")
    return pl.kernel(
        functools.partial(_seg_body, n_t=n_t, n_b=n_b,
                          segs_per_w=segs_per_w, nc=nc),
        mesh=mesh,
        out_type=jax.ShapeDtypeStruct((n_b * n_t * d,), jnp.float32),
        scratch_types=[
            pltpu.VMEM((n_b,), jnp.int32),
            pltpu.VMEM((d * _LANES,), jnp.float32),
            pltpu.VMEM((chunk_w,), jnp.float32),
            pltpu.VMEM((chunk_w,), jnp.float32),
            pltpu.SemaphoreType.DMA,
            pltpu.SemaphoreType.DMA,
        ],
        compiler_params=pltpu.CompilerParams(use_tc_tiling_on_sc=False,
                                             needs_layout_passes=False),
    )


def kernel(inputs, embedding_table):
    n_b, n_t = inputs.shape
    v, d = embedding_table.shape
    idx_t = inputs.T.astype(jnp.int32)
    tab_t = jnp.pad(embedding_table.T.astype(jnp.float32),
                    ((0, 0), (0, _LANES - v))).reshape(-1)
    flat = _make_sc_kernel(n_b, n_t, d)(idx_t, tab_t)
    out5 = flat.reshape(n_t, d // 8, n_b // 128, 8, 128)
    return out5.transpose(2, 4, 0, 1, 3).reshape(n_b, n_t, d)


# speedup vs baseline: 92.4936x; 1.2110x over previous
import functools

import jax
import jax.numpy as jnp
from jax import lax
from jax.experimental import pallas as pl
from jax.experimental.pallas import tpu as pltpu
from jax.experimental.pallas import tpu_sc as plsc

_LANES = 16


def _seg_body(idx_hbm, tab_hbm, out_hbm, idx_v, tab_v, st0, st1, sem0, sem1,
              *, n_t, n_b, segs_per_w, nc):
    n_bt = n_b // 128
    chunk_bt = 16
    chunk_w = chunk_bt * 1024
    n_chunks = n_bt // chunk_bt
    groups = chunk_w // 128

    wid = lax.axis_index("s") * nc + lax.axis_index("c")
    pltpu.sync_copy(tab_hbm, tab_v)
    s0 = wid * segs_per_w

    def seg_loop(sl, prev_t):
        s = s0 + sl
        t = s // 8
        dt = s % 8

        @pl.when(t != prev_t)
        def _():
            pltpu.sync_copy(idx_hbm.at[t], idx_v)
        seg_base = t * (64 * n_b) + dt * (8 * n_b)
        d_base = dt * 128
        zero16 = jnp.zeros((_LANES,), jnp.int32)
        bvecs = [zero16 + (d_base + di * _LANES) for di in range(8)]

        def pair_loop(p, carry2):
            for b, st, sem in ((0, st0, sem0), (1, st1, sem1)):
                c = p * 2 + b

                @pl.when((sl > 0) | (p > 0))
                def _():
                    pltpu.make_async_copy(
                        st, out_hbm.at[pl.ds(0, chunk_w)], sem).wait()

                @plsc.parallel_loop(0, groups, unroll=4)
                def _(g):
                    gg = c * groups + g
                    idxv = idx_v[pl.ds(gg * _LANES, _LANES)]
                    off0 = (g // 8) * 1024 + (g % 8) * _LANES
                    for di in range(8):
                        val = plsc.load_gather(tab_v, [idxv + bvecs[di]])
                        st[pl.ds(off0 + di * 128, _LANES)] = val
                pltpu.async_copy(
                    st, out_hbm.at[pl.ds(seg_base + c * chunk_w, chunk_w)],
                    sem)
            return carry2

        lax.fori_loop(0, n_chunks // 2, pair_loop, 0)
        return t

    lax.fori_loop(0, segs_per_w, seg_loop, jnp.int32(-1))
    for st, sem in ((st0, sem0), (st1, sem1)):
        pltpu.make_async_copy(st, out_hbm.at[pl.ds(0, chunk_w)], sem).wait()


def _make_sc_kernel(n_b, n_t, d):
    info = plsc.get_sparse_core_info()
    nc, ns = info.num_cores, info.num_subcores
    nw = nc * ns
    n_seg = n_t * (d // 8)
    assert n_seg % nw == 0 and n_b % 2048 == 0
    segs_per_w = n_seg // nw
    chunk_w = 16 * 1024

    mesh = plsc.VectorSubcoreMesh(core_axis_name="c", subcore_axis_name="s")
    return pl.kernel(
        functools.partial(_seg_body, n_t=n_t, n_b=n_b,
                          segs_per_w=segs_per_w, nc=nc),
        mesh=mesh,
        out_type=jax.ShapeDtypeStruct((n_b * n_t * d,), jnp.float32),
        scratch_types=[
            pltpu.VMEM((n_b,), jnp.int32),
            pltpu.VMEM((d * _LANES,), jnp.float32),
            pltpu.VMEM((chunk_w,), jnp.float32),
            pltpu.VMEM((chunk_w,), jnp.float32),
            pltpu.SemaphoreType.DMA,
            pltpu.SemaphoreType.DMA,
        ],
        compiler_params=pltpu.CompilerParams(use_tc_tiling_on_sc=False,
                                             needs_layout_passes=False),
    )


def kernel(inputs, embedding_table):
    n_b, n_t = inputs.shape
    v, d = embedding_table.shape
    idx_t = inputs.T.astype(jnp.int32)
    tab_t = jnp.pad(embedding_table.T.astype(jnp.float32),
                    ((0, 0), (0, _LANES - v))).reshape(-1)
    flat = _make_sc_kernel(n_b, n_t, d)(idx_t, tab_t)
    out5 = flat.reshape(n_t, d // 8, n_b // 128, 8, 128)
    return out5.transpose(2, 4, 0, 1, 3).reshape(n_b, n_t, d)


# speedup vs baseline: 93.1092x; 1.0067x over previous
import functools

import jax
import jax.numpy as jnp
from jax import lax
from jax.experimental import pallas as pl
from jax.experimental.pallas import tpu as pltpu
from jax.experimental.pallas import tpu_sc as plsc

_LANES = 16


def _seg_body(idx_hbm, tab_hbm, out_hbm, idx_v, tab_v, st0, st1, sem0, sem1,
              *, n_t, n_b, segs_per_w, nc):
    n_bt = n_b // 128
    chunk_bt = 32
    chunk_w = chunk_bt * 1024
    n_chunks = n_bt // chunk_bt
    groups = chunk_w // 128

    wid = lax.axis_index("s") * nc + lax.axis_index("c")
    pltpu.sync_copy(tab_hbm, tab_v)
    s0 = wid * segs_per_w

    def seg_loop(sl, prev_t):
        s = s0 + sl
        t = s // 8
        dt = s % 8

        @pl.when(t != prev_t)
        def _():
            pltpu.sync_copy(idx_hbm.at[t], idx_v)
        seg_base = t * (64 * n_b) + dt * (8 * n_b)
        d_base = dt * 128
        zero16 = jnp.zeros((_LANES,), jnp.int32)
        bvecs = [zero16 + (d_base + di * _LANES) for di in range(8)]

        def pair_loop(p, carry2):
            for b, st, sem in ((0, st0, sem0), (1, st1, sem1)):
                c = p * 2 + b

                @pl.when((sl > 0) | (p > 0))
                def _():
                    pltpu.make_async_copy(
                        st, out_hbm.at[pl.ds(0, chunk_w)], sem).wait()

                @plsc.parallel_loop(0, groups, unroll=4)
                def _(g):
                    gg = c * groups + g
                    idxv = idx_v[pl.ds(gg * _LANES, _LANES)]
                    off0 = (g // 8) * 1024 + (g % 8) * _LANES
                    for di in range(8):
                        val = plsc.load_gather(tab_v, [idxv + bvecs[di]])
                        st[pl.ds(off0 + di * 128, _LANES)] = val
                pltpu.async_copy(
                    st, out_hbm.at[pl.ds(seg_base + c * chunk_w, chunk_w)],
                    sem)
            return carry2

        lax.fori_loop(0, n_chunks // 2, pair_loop, 0)
        return t

    lax.fori_loop(0, segs_per_w, seg_loop, jnp.int32(-1))
    for st, sem in ((st0, sem0), (st1, sem1)):
        pltpu.make_async_copy(st, out_hbm.at[pl.ds(0, chunk_w)], sem).wait()


def _make_sc_kernel(n_b, n_t, d):
    info = plsc.get_sparse_core_info()
    nc, ns = info.num_cores, info.num_subcores
    nw = nc * ns
    n_seg = n_t * (d // 8)
    assert n_seg % nw == 0 and n_b % 2048 == 0
    segs_per_w = n_seg // nw
    chunk_w = 32 * 1024

    mesh = plsc.VectorSubcoreMesh(core_axis_name="c", subcore_axis_name="s")
    return pl.kernel(
        functools.partial(_seg_body, n_t=n_t, n_b=n_b,
                          segs_per_w=segs_per_w, nc=nc),
        mesh=mesh,
        out_type=jax.ShapeDtypeStruct((n_b * n_t * d,), jnp.float32),
        scratch_types=[
            pltpu.VMEM((n_b,), jnp.int32),
            pltpu.VMEM((d * _LANES,), jnp.float32),
            pltpu.VMEM((chunk_w,), jnp.float32),
            pltpu.VMEM((chunk_w,), jnp.float32),
            pltpu.SemaphoreType.DMA,
            pltpu.SemaphoreType.DMA,
        ],
        compiler_params=pltpu.CompilerParams(use_tc_tiling_on_sc=False,
                                             needs_layout_passes=False),
    )


def kernel(inputs, embedding_table):
    n_b, n_t = inputs.shape
    v, d = embedding_table.shape
    idx_t = inputs.T.astype(jnp.int32)
    tab_t = jnp.pad(embedding_table.T.astype(jnp.float32),
                    ((0, 0), (0, _LANES - v))).reshape(-1)
    flat = _make_sc_kernel(n_b, n_t, d)(idx_t, tab_t)
    out5 = flat.reshape(n_t, d // 8, n_b // 128, 8, 128)
    return out5.transpose(2, 4, 0, 1, 3).reshape(n_b, n_t, d)


# speedup vs baseline: 93.4785x; 1.0040x over previous
import functools

import jax
import jax.numpy as jnp
from jax import lax
from jax.experimental import pallas as pl
from jax.experimental.pallas import tpu as pltpu
from jax.experimental.pallas import tpu_sc as plsc

_LANES = 16


def _seg_body(idx_hbm, tab_hbm, out_hbm, idx_v, tab_v, st0, st1, sem0, sem1,
              *, n_t, n_b, segs_per_w, nc):
    n_bt = n_b // 128
    chunk_bt = 32
    chunk_w = chunk_bt * 1024
    n_chunks = n_bt // chunk_bt
    groups = chunk_w // 128

    wid = lax.axis_index("s") * nc + lax.axis_index("c")
    pltpu.sync_copy(tab_hbm, tab_v)
    s0 = wid * segs_per_w

    def seg_loop(sl, prev_t):
        s = s0 + sl
        t = s // 8
        dt = s % 8

        @pl.when(t != prev_t)
        def _():
            pltpu.sync_copy(idx_hbm.at[t], idx_v)
        seg_base = t * (64 * n_b) + dt * (8 * n_b)
        d_base = dt * 128
        zero16 = jnp.zeros((_LANES,), jnp.int32)
        bvecs = [zero16 + (d_base + di * _LANES) for di in range(8)]

        def pair_loop(p, carry2):
            for b, st, sem in ((0, st0, sem0), (1, st1, sem1)):
                c = p * 2 + b

                @pl.when((sl > 0) | (p > 0))
                def _():
                    pltpu.make_async_copy(
                        st, out_hbm.at[pl.ds(0, chunk_w)], sem).wait()

                @plsc.parallel_loop(0, groups, unroll=8)
                def _(g):
                    gg = c * groups + g
                    idxv = idx_v[pl.ds(gg * _LANES, _LANES)]
                    off0 = (g // 8) * 1024 + (g % 8) * _LANES
                    for di in range(8):
                        val = plsc.load_gather(tab_v, [idxv + bvecs[di]])
                        st[pl.ds(off0 + di * 128, _LANES)] = val
                pltpu.async_copy(
                    st, out_hbm.at[pl.ds(seg_base + c * chunk_w, chunk_w)],
                    sem)
            return carry2

        lax.fori_loop(0, n_chunks // 2, pair_loop, 0)
        return t

    lax.fori_loop(0, segs_per_w, seg_loop, jnp.int32(-1))
    for st, sem in ((st0, sem0), (st1, sem1)):
        pltpu.make_async_copy(st, out_hbm.at[pl.ds(0, chunk_w)], sem).wait()


def _make_sc_kernel(n_b, n_t, d):
    info = plsc.get_sparse_core_info()
    nc, ns = info.num_cores, info.num_subcores
    nw = nc * ns
    n_seg = n_t * (d // 8)
    assert n_seg % nw == 0 and n_b % 2048 == 0
    segs_per_w = n_seg // nw
    chunk_w = 32 * 1024

    mesh = plsc.VectorSubcoreMesh(core_axis_name="c", subcore_axis_name="s")
    return pl.kernel(
        functools.partial(_seg_body, n_t=n_t, n_b=n_b,
                          segs_per_w=segs_per_w, nc=nc),
        mesh=mesh,
        out_type=jax.ShapeDtypeStruct((n_b * n_t * d,), jnp.float32),
        scratch_types=[
            pltpu.VMEM((n_b,), jnp.int32),
            pltpu.VMEM((d * _LANES,), jnp.float32),
            pltpu.VMEM((chunk_w,), jnp.float32),
            pltpu.VMEM((chunk_w,), jnp.float32),
            pltpu.SemaphoreType.DMA,
            pltpu.SemaphoreType.DMA,
        ],
        compiler_params=pltpu.CompilerParams(use_tc_tiling_on_sc=False,
                                             needs_layout_passes=False),
    )


def kernel(inputs, embedding_table):
    n_b, n_t = inputs.shape
    v, d = embedding_table.shape
    idx_t = inputs.T.astype(jnp.int32)
    tab_t = jnp.pad(embedding_table.T.astype(jnp.float32),
                    ((0, 0), (0, _LANES - v))).reshape(-1)
    flat = _make_sc_kernel(n_b, n_t, d)(idx_t, tab_t)
    out5 = flat.reshape(n_t, d // 8, n_b // 128, 8, 128)
    return out5.transpose(2, 4, 0, 1, 3).reshape(n_b, n_t, d)
